# baseline (speedup 1.0000x reference)
import functools

import jax
import jax.numpy as jnp
import numpy as np
from jax import lax
from jax.experimental import pallas as pl
from jax.experimental.pallas import tpu as pltpu
from jax.experimental.pallas import tpu_sc as plsc

ROWS, LANES = 2048, 128
P = ROWS * LANES
B = 8
NT = 16
CHUNK = P // NT
NV = CHUNK // 16
IMGS_PER_CORE = 4


def _pack_body(logits_ref, target_ref, keys_ref):
    lab = target_ref[0]
    labf = lab.astype(jnp.float32)
    e = 1.0 - logits_ref[0] * (2.0 * labf - 1.0)
    epos = jnp.maximum(e, 0.0)
    bits = (lax.bitcast_convert_type(epos, jnp.int32) & jnp.int32(~1)) | lab
    keys_ref[0] = bits


def _pack(lg, tg):
    return pl.pallas_call(
        _pack_body,
        grid=(B,),
        in_specs=[
            pl.BlockSpec((1, ROWS, LANES), lambda b: (b, 0, 0)),
            pl.BlockSpec((1, ROWS, LANES), lambda b: (b, 0, 0)),
        ],
        out_specs=pl.BlockSpec((1, ROWS, LANES), lambda b: (b, 0, 0)),
        out_shape=jax.ShapeDtypeStruct((B, ROWS, LANES), jnp.int32),
    )(lg, tg)


def _take16(vec, idx):
    return lax.gather(
        vec, idx[:, None],
        dimension_numbers=lax.GatherDimensionNumbers(
            offset_dims=(), collapsed_slice_dims=(0,), start_index_map=(0,)),
        slice_sizes=(1,),
        mode=lax.GatherScatterMode.PROMISE_IN_BOUNDS)


def _sc_sort(keys):
    mesh = plsc.VectorSubcoreMesh(core_axis_name="c", subcore_axis_name="s")

    @functools.partial(
        pl.kernel,
        mesh=mesh,
        compiler_params=pltpu.CompilerParams(use_tc_tiling_on_sc=False,
                                             needs_layout_passes=False),
        out_type=jax.ShapeDtypeStruct((B, P), jnp.int32),
        scratch_types=[
            pltpu.VMEM((CHUNK,), jnp.int32),
            pltpu.VMEM((CHUNK,), jnp.int32),
            pltpu.VMEM((CHUNK,), jnp.int32),
            pltpu.VMEM((16, 256), jnp.int32),
            pltpu.VMEM((16, 256), jnp.int32),
            pltpu.VMEM((256,), jnp.int32),
            pltpu.VMEM((256,), jnp.int32),
            pltpu.VMEM((NT, 256), jnp.int32),
            pltpu.VMEM_SHARED((P,), jnp.int32),
            pltpu.VMEM_SHARED((P,), jnp.int32),
            pltpu.VMEM_SHARED((NT, 256), jnp.int32),
            pltpu.SemaphoreType.DMA,
        ],
    )
    def k(keys_hbm, out_hbm, buf, valbuf, posbuf, cnt2, pfx2, offs,
          hist, hists_local, S0, S1, hist_sh, sem):
        c = lax.axis_index("c")
        t = lax.axis_index("s")
        lane = lax.iota(jnp.int32, 16)
        zeros16 = jnp.zeros((16,), jnp.int32)
        lane_nv = lane * NV
        my = pl.ds(t * CHUNK, CHUNK)

        for l in range(16):
            for ch in range(16):
                cnt2[l, pl.ds(ch * 16, 16)] = zeros16

        def do_pass(shift, dst, last=False):
            def sweep(q, _):
                sl = pl.ds(q * 16, 16)
                v = buf[sl]
                d = 255 - ((v >> shift) & 255)
                cg = plsc.load_gather(cnt2, [lane, d])
                plsc.store_scatter(cnt2, [lane, d], cg + 1)
                posbuf[sl] = cg
                valbuf[sl] = v
                return 0

            lax.fori_loop(0, NV, sweep, 0, unroll=8)

            for ch in range(16):
                chs = pl.ds(ch * 16, 16)
                acc = zeros16
                for l in range(16):
                    rowv = cnt2[l, chs]
                    pfx2[l, chs] = acc
                    cnt2[l, chs] = zeros16
                    acc = acc + rowv
                hist[chs] = acc

            pltpu.sync_copy(hist, hist_sh.at[t])
            plsc.subcore_barrier()
            pltpu.sync_copy(hist_sh, hists_local)

            carry = jnp.int32(0)
            for ch in range(16):
                col = zeros16
                part = zeros16
                for tt in range(16):
                    h = hists_local[tt, pl.ds(ch * 16, 16)]
                    col = col + h
                    tv = jnp.full((16,), tt, jnp.int32)
                    part = part + jnp.where(tv < t, h, zeros16)
                incl = plsc.cumsum(col)
                ov = (incl - col) + carry + part
                carry = carry + jnp.sum(col)
                chs = pl.ds(ch * 16, 16)
                for l in range(16):
                    pfx2[l, chs] = pfx2[l, chs] + ov

            def fix(q, _):
                sl = pl.ds(q * 16, 16)
                dv = 255 - ((valbuf[sl] >> shift) & 255)
                pos = posbuf[sl] + plsc.load_gather(pfx2, [lane, dv])
                if not last:
                    pos = ((pos & ~jnp.int32(16383)) | ((pos & 1023) << 4)
                           | ((pos >> 10) & 15))
                posbuf[sl] = pos
                return 0

            lax.fori_loop(0, NV, fix, 0, unroll=8)
            pltpu.async_copy(valbuf, dst.at[posbuf], sem).wait()
            plsc.subcore_barrier()

        def img_body(ii, _):
            img = c * IMGS_PER_CORE + ii
            pltpu.sync_copy(keys_hbm.at[img, my], buf)
            do_pass(0, S0)
            pltpu.sync_copy(S0.at[my], buf)
            do_pass(8, S1)
            pltpu.sync_copy(S1.at[my], buf)
            do_pass(16, S0)
            pltpu.sync_copy(S0.at[my], buf)
            do_pass(24, S1, last=True)
            pltpu.sync_copy(S1.at[my], out_hbm.at[img, my])
            plsc.subcore_barrier()
            return 0

        lax.fori_loop(0, IMGS_PER_CORE, img_body, 0)

    return k(keys)


def _eval_body(skeys_ref, out_ref):
    b = pl.program_id(0)
    row = lax.broadcasted_iota(jnp.int32, (ROWS, LANES), 0)
    lane = lax.broadcasted_iota(jnp.int32, (ROWS, LANES), 1)

    sbits = skeys_ref[0]
    l_sorted = (sbits & 1).astype(jnp.float32)
    e_sorted = lax.bitcast_convert_type(sbits & jnp.int32(~1), jnp.float32)
    G = jnp.sum(l_sorted)

    cs = l_sorted
    for sh in (1, 2, 4, 8, 16, 32, 64):
        cs = cs + jnp.where(lane >= sh, pltpu.roll(cs, sh, axis=1), 0.0)
    rt = cs[:, LANES - 1:LANES]
    rs = rt
    rowv = lax.broadcasted_iota(jnp.int32, (ROWS, 1), 0)
    for sh in (1, 2, 4, 8, 16, 32, 64, 128, 256, 512, 1024):
        rs = rs + jnp.where(rowv >= sh, pltpu.roll(rs, sh, axis=0), 0.0)
    c = cs + (rs - rt)

    k = (row * LANES + lane).astype(jnp.float32) + 1.0
    n = k - c
    cm1 = c - l_sorted
    nm1 = (k - 1.0) - cm1
    jk = k / (G + n)
    jm1 = (k - 1.0) / jnp.maximum(G + nm1, 1.0)
    loss = jnp.sum(e_sorted * (jk - jm1))

    prev = jnp.where(b == 0, 0.0, out_ref[0, 0])
    out_ref[0, 0] = prev + loss * (1.0 / B)


def _eval(skeys):
    out = pl.pallas_call(
        _eval_body,
        grid=(B,),
        in_specs=[pl.BlockSpec((1, ROWS, LANES), lambda b: (b, 0, 0))],
        out_specs=pl.BlockSpec((1, 1), lambda b: (0, 0),
                               memory_space=pltpu.SMEM),
        out_shape=jax.ShapeDtypeStruct((1, 1), jnp.float32),
    )(skeys)
    return out.reshape(())


@jax.jit
def _run(logits, target):
    lg = logits.reshape(B, ROWS, LANES)
    tg = target.reshape(B, ROWS, LANES)
    keys = _pack(lg, tg).reshape(B, P)
    skeys = _sc_sort(keys)
    return _eval(skeys.reshape(B, ROWS, LANES))


def kernel(logits, target):
    return _run(logits, target)


# speedup vs baseline: 1.1514x; 1.1514x over previous
import functools

import jax
import jax.numpy as jnp
import numpy as np
from jax import lax
from jax.experimental import pallas as pl
from jax.experimental.pallas import tpu as pltpu
from jax.experimental.pallas import tpu_sc as plsc

ROWS, LANES = 2048, 128
P = ROWS * LANES
B = 8
NT = 16
CHUNK = P // NT
NV = CHUNK // 16
IMGS_PER_CORE = 4


def _pack_body(logits_ref, target_ref, keys_ref):
    lab = target_ref[0]
    labf = lab.astype(jnp.float32)
    e = 1.0 - logits_ref[0] * (2.0 * labf - 1.0)
    epos = jnp.maximum(e, 0.0)
    bits = (lax.bitcast_convert_type(epos, jnp.int32) & jnp.int32(~1)) | lab
    keys_ref[0] = bits


def _pack(lg, tg):
    return pl.pallas_call(
        _pack_body,
        grid=(B,),
        in_specs=[
            pl.BlockSpec((1, ROWS, LANES), lambda b: (b, 0, 0)),
            pl.BlockSpec((1, ROWS, LANES), lambda b: (b, 0, 0)),
        ],
        out_specs=pl.BlockSpec((1, ROWS, LANES), lambda b: (b, 0, 0)),
        out_shape=jax.ShapeDtypeStruct((B, ROWS, LANES), jnp.int32),
    )(lg, tg)


def _take16(vec, idx):
    return lax.gather(
        vec, idx[:, None],
        dimension_numbers=lax.GatherDimensionNumbers(
            offset_dims=(), collapsed_slice_dims=(0,), start_index_map=(0,)),
        slice_sizes=(1,),
        mode=lax.GatherScatterMode.PROMISE_IN_BOUNDS)


def _sc_sort(keys):
    mesh = plsc.VectorSubcoreMesh(core_axis_name="c", subcore_axis_name="s")

    @functools.partial(
        pl.kernel,
        mesh=mesh,
        compiler_params=pltpu.CompilerParams(use_tc_tiling_on_sc=False,
                                             needs_layout_passes=False),
        out_type=jax.ShapeDtypeStruct((B, P), jnp.int32),
        scratch_types=[
            pltpu.VMEM((CHUNK,), jnp.int32),
            pltpu.VMEM((CHUNK,), jnp.int32),
            pltpu.VMEM((CHUNK,), jnp.int32),
            pltpu.VMEM((16, 256), jnp.int32),
            pltpu.VMEM((16, 256), jnp.int32),
            pltpu.VMEM((256,), jnp.int32),
            pltpu.VMEM((256,), jnp.int32),
            pltpu.VMEM((NT, 256), jnp.int32),
            pltpu.VMEM_SHARED((P,), jnp.int32),
            pltpu.VMEM_SHARED((P,), jnp.int32),
            pltpu.VMEM_SHARED((NT, 256), jnp.int32),
            pltpu.SemaphoreType.DMA,
        ],
    )
    def k(keys_hbm, out_hbm, buf, valbuf, posbuf, cnt2, pfx2, offs,
          hist, hists_local, S0, S1, hist_sh, sem):
        c = lax.axis_index("c")
        t = lax.axis_index("s")
        lane = lax.iota(jnp.int32, 16)
        zeros16 = jnp.zeros((16,), jnp.int32)
        lane_nv = lane * NV
        my = pl.ds(t * CHUNK, CHUNK)

        for l in range(16):
            for ch in range(16):
                cnt2[l, pl.ds(ch * 16, 16)] = zeros16

        def do_pass(shift, dst, last=False):
            def sweep(q, _):
                sl = pl.ds(q * 16, 16)
                v = buf[sl]
                d = 255 - ((v >> shift) & 255)
                cg = plsc.load_gather(cnt2, [lane, d])
                plsc.store_scatter(cnt2, [lane, d], cg + 1)
                posbuf[sl] = cg
                valbuf[sl] = v
                return 0

            lax.fori_loop(0, NV, sweep, 0)

            for ch in range(16):
                chs = pl.ds(ch * 16, 16)
                acc = zeros16
                for l in range(16):
                    rowv = cnt2[l, chs]
                    pfx2[l, chs] = acc
                    cnt2[l, chs] = zeros16
                    acc = acc + rowv
                hist[chs] = acc

            pltpu.sync_copy(hist, hist_sh.at[t])
            plsc.subcore_barrier()
            pltpu.sync_copy(hist_sh, hists_local)

            carry = jnp.int32(0)
            for ch in range(16):
                col = zeros16
                part = zeros16
                for tt in range(16):
                    h = hists_local[tt, pl.ds(ch * 16, 16)]
                    col = col + h
                    tv = jnp.full((16,), tt, jnp.int32)
                    part = part + jnp.where(tv < t, h, zeros16)
                incl = plsc.cumsum(col)
                ov = (incl - col) + carry + part
                carry = carry + jnp.sum(col)
                chs = pl.ds(ch * 16, 16)
                for l in range(16):
                    pfx2[l, chs] = pfx2[l, chs] + ov

            def fix(q, _):
                sl = pl.ds(q * 16, 16)
                dv = 255 - ((valbuf[sl] >> shift) & 255)
                pos = posbuf[sl] + plsc.load_gather(pfx2, [lane, dv])
                if not last:
                    pos = ((pos & ~jnp.int32(16383)) | ((pos & 1023) << 4)
                           | ((pos >> 10) & 15))
                posbuf[sl] = pos
                return 0

            lax.fori_loop(0, NV, fix, 0)
            pltpu.async_copy(valbuf, dst.at[posbuf], sem).wait()
            plsc.subcore_barrier()

        def img_body(ii, _):
            img = c * IMGS_PER_CORE + ii
            pltpu.sync_copy(keys_hbm.at[img, my], buf)
            do_pass(0, S0)
            pltpu.sync_copy(S0.at[my], buf)
            do_pass(8, S1)
            pltpu.sync_copy(S1.at[my], buf)
            do_pass(16, S0)
            pltpu.sync_copy(S0.at[my], buf)
            do_pass(24, S1, last=True)
            pltpu.sync_copy(S1.at[my], out_hbm.at[img, my])
            plsc.subcore_barrier()
            return 0

        lax.fori_loop(0, IMGS_PER_CORE, img_body, 0)

    return k(keys)


def _eval_body(skeys_ref, out_ref):
    b = pl.program_id(0)
    row = lax.broadcasted_iota(jnp.int32, (ROWS, LANES), 0)
    lane = lax.broadcasted_iota(jnp.int32, (ROWS, LANES), 1)

    sbits = skeys_ref[0]
    l_sorted = (sbits & 1).astype(jnp.float32)
    e_sorted = lax.bitcast_convert_type(sbits & jnp.int32(~1), jnp.float32)
    G = jnp.sum(l_sorted)

    cs = l_sorted
    for sh in (1, 2, 4, 8, 16, 32, 64):
        cs = cs + jnp.where(lane >= sh, pltpu.roll(cs, sh, axis=1), 0.0)
    rt = cs[:, LANES - 1:LANES]
    rs = rt
    rowv = lax.broadcasted_iota(jnp.int32, (ROWS, 1), 0)
    for sh in (1, 2, 4, 8, 16, 32, 64, 128, 256, 512, 1024):
        rs = rs + jnp.where(rowv >= sh, pltpu.roll(rs, sh, axis=0), 0.0)
    c = cs + (rs - rt)

    k = (row * LANES + lane).astype(jnp.float32) + 1.0
    n = k - c
    cm1 = c - l_sorted
    nm1 = (k - 1.0) - cm1
    jk = k / (G + n)
    jm1 = (k - 1.0) / jnp.maximum(G + nm1, 1.0)
    loss = jnp.sum(e_sorted * (jk - jm1))

    prev = jnp.where(b == 0, 0.0, out_ref[0, 0])
    out_ref[0, 0] = prev + loss * (1.0 / B)


def _eval(skeys):
    out = pl.pallas_call(
        _eval_body,
        grid=(B,),
        in_specs=[pl.BlockSpec((1, ROWS, LANES), lambda b: (b, 0, 0))],
        out_specs=pl.BlockSpec((1, 1), lambda b: (0, 0),
                               memory_space=pltpu.SMEM),
        out_shape=jax.ShapeDtypeStruct((1, 1), jnp.float32),
    )(skeys)
    return out.reshape(())


@jax.jit
def _run(logits, target):
    lg = logits.reshape(B, ROWS, LANES)
    tg = target.reshape(B, ROWS, LANES)
    keys = _pack(lg, tg).reshape(B, P)
    skeys = _sc_sort(keys)
    return _eval(skeys.reshape(B, ROWS, LANES))


def kernel(logits, target):
    return _run(logits, target)


# speedup vs baseline: 1.3187x; 1.1453x over previous
import functools

import jax
import jax.numpy as jnp
import numpy as np
from jax import lax
from jax.experimental import pallas as pl
from jax.experimental.pallas import tpu as pltpu
from jax.experimental.pallas import tpu_sc as plsc

ROWS, LANES = 2048, 128
P = ROWS * LANES
B = 8
NT = 16
CHUNK = P // NT
NV = CHUNK // 16
IMGS_PER_CORE = 4


def _pack_body(logits_ref, target_ref, keys_ref):
    lab = target_ref[0]
    labf = lab.astype(jnp.float32)
    e = 1.0 - logits_ref[0] * (2.0 * labf - 1.0)
    epos = jnp.maximum(e, 0.0)
    bits = (lax.bitcast_convert_type(epos, jnp.int32) & jnp.int32(~1)) | lab
    keys_ref[0] = bits


def _pack(lg, tg):
    return pl.pallas_call(
        _pack_body,
        grid=(B,),
        in_specs=[
            pl.BlockSpec((1, ROWS, LANES), lambda b: (b, 0, 0)),
            pl.BlockSpec((1, ROWS, LANES), lambda b: (b, 0, 0)),
        ],
        out_specs=pl.BlockSpec((1, ROWS, LANES), lambda b: (b, 0, 0)),
        out_shape=jax.ShapeDtypeStruct((B, ROWS, LANES), jnp.int32),
    )(lg, tg)


def _take16(vec, idx):
    return lax.gather(
        vec, idx[:, None],
        dimension_numbers=lax.GatherDimensionNumbers(
            offset_dims=(), collapsed_slice_dims=(0,), start_index_map=(0,)),
        slice_sizes=(1,),
        mode=lax.GatherScatterMode.PROMISE_IN_BOUNDS)


def _sc_sort(keys):
    mesh = plsc.VectorSubcoreMesh(core_axis_name="c", subcore_axis_name="s")

    NB = 2048
    PH = 256

    @functools.partial(
        pl.kernel,
        mesh=mesh,
        compiler_params=pltpu.CompilerParams(use_tc_tiling_on_sc=False,
                                             needs_layout_passes=False),
        out_type=jax.ShapeDtypeStruct((B, P), jnp.int32),
        scratch_types=[
            pltpu.VMEM((CHUNK,), jnp.int32),
            pltpu.VMEM((CHUNK,), jnp.int32),
            pltpu.VMEM((16, NB), jnp.int32),
            pltpu.VMEM((NB,), jnp.int32),
            pltpu.VMEM((NT, PH), jnp.int32),
            pltpu.VMEM_SHARED((P,), jnp.int32),
            pltpu.VMEM_SHARED((P,), jnp.int32),
            pltpu.VMEM_SHARED((NT, PH), jnp.int32),
            pltpu.SemaphoreType.DMA,
        ],
    )
    def k(keys_hbm, out_hbm, buf, posbuf, tbl,
          hist, hists_local, S0, S1, hist_sh, sem):
        c = lax.axis_index("c")
        t = lax.axis_index("s")
        lane = lax.iota(jnp.int32, 16)
        zeros16 = jnp.zeros((16,), jnp.int32)
        my = pl.ds(t * CHUNK, CHUNK)

        def zero_tbl(ch, _):
            chs = pl.ds(ch * 16, 16)
            for l in range(16):
                tbl[l, chs] = zeros16
            return 0

        lax.fori_loop(0, NB // 16, zero_tbl, 0)

        def do_pass(shift, nbits, dst, last=False):
            nb = 1 << nbits
            comp = nb - 1

            def sweep(q, _):
                sl = pl.ds(q * 16, 16)
                d = comp - ((buf[sl] >> shift) & comp)
                cg = plsc.load_gather(tbl, [lane, d])
                plsc.store_scatter(tbl, [lane, d], cg + 1)
                posbuf[sl] = cg
                return 0

            lax.fori_loop(0, NV, sweep, 0)

            def pfx_chunk(ch, _):
                chs = pl.ds(ch * 16, 16)
                acc = zeros16
                for l in range(16):
                    rowv = tbl[l, chs]
                    tbl[l, chs] = acc
                    acc = acc + rowv
                hist[chs] = acc
                return 0

            lax.fori_loop(0, nb // 16, pfx_chunk, 0)

            def phase(ph, carry):
                pltpu.sync_copy(hist.at[pl.ds(ph * PH, PH)], hist_sh.at[t])
                plsc.subcore_barrier()
                pltpu.sync_copy(hist_sh, hists_local)

                def scan_chunk(ch, carry):
                    col = zeros16
                    part = zeros16
                    for tt in range(16):
                        h = hists_local[tt, pl.ds(ch * 16, 16)]
                        col = col + h
                        tv = jnp.full((16,), tt, jnp.int32)
                        part = part + jnp.where(tv < t, h, zeros16)
                    incl = plsc.cumsum(col)
                    ov = (incl - col) + carry + part
                    chs = pl.ds(ph * PH + ch * 16, 16)
                    for l in range(16):
                        tbl[l, chs] = tbl[l, chs] + ov
                    return carry + jnp.sum(col)

                carry = lax.fori_loop(0, PH // 16, scan_chunk, carry)
                plsc.subcore_barrier()
                return carry

            lax.fori_loop(0, nb // PH, phase, jnp.int32(0))

            def fix(q, _):
                sl = pl.ds(q * 16, 16)
                dv = comp - ((buf[sl] >> shift) & comp)
                pos = posbuf[sl] + plsc.load_gather(tbl, [lane, dv])
                if not last:
                    pos = ((pos & ~jnp.int32(16383)) | ((pos & 1023) << 4)
                           | ((pos >> 10) & 15))
                posbuf[sl] = pos
                return 0

            lax.fori_loop(0, NV, fix, 0)

            def rezero(ch, _):
                chs = pl.ds(ch * 16, 16)
                for l in range(16):
                    tbl[l, chs] = zeros16
                return 0

            lax.fori_loop(0, nb // 16, rezero, 0)
            pltpu.async_copy(buf, dst.at[posbuf], sem).wait()
            plsc.subcore_barrier()

        def img_body(ii, _):
            img = c * IMGS_PER_CORE + ii
            pltpu.sync_copy(keys_hbm.at[img, my], buf)
            do_pass(0, 11, S0)
            pltpu.sync_copy(S0.at[my], buf)
            do_pass(11, 10, S1)
            pltpu.sync_copy(S1.at[my], buf)
            do_pass(21, 10, S0, last=True)
            pltpu.sync_copy(S0.at[my], out_hbm.at[img, my])
            plsc.subcore_barrier()
            return 0

        lax.fori_loop(0, IMGS_PER_CORE, img_body, 0)

    return k(keys)


def _eval_body(skeys_ref, out_ref):
    b = pl.program_id(0)
    row = lax.broadcasted_iota(jnp.int32, (ROWS, LANES), 0)
    lane = lax.broadcasted_iota(jnp.int32, (ROWS, LANES), 1)

    sbits = skeys_ref[0]
    l_sorted = (sbits & 1).astype(jnp.float32)
    e_sorted = lax.bitcast_convert_type(sbits & jnp.int32(~1), jnp.float32)
    G = jnp.sum(l_sorted)

    cs = l_sorted
    for sh in (1, 2, 4, 8, 16, 32, 64):
        cs = cs + jnp.where(lane >= sh, pltpu.roll(cs, sh, axis=1), 0.0)
    rt = cs[:, LANES - 1:LANES]
    rs = rt
    rowv = lax.broadcasted_iota(jnp.int32, (ROWS, 1), 0)
    for sh in (1, 2, 4, 8, 16, 32, 64, 128, 256, 512, 1024):
        rs = rs + jnp.where(rowv >= sh, pltpu.roll(rs, sh, axis=0), 0.0)
    c = cs + (rs - rt)

    k = (row * LANES + lane).astype(jnp.float32) + 1.0
    n = k - c
    cm1 = c - l_sorted
    nm1 = (k - 1.0) - cm1
    jk = k / (G + n)
    jm1 = (k - 1.0) / jnp.maximum(G + nm1, 1.0)
    loss = jnp.sum(e_sorted * (jk - jm1))

    prev = jnp.where(b == 0, 0.0, out_ref[0, 0])
    out_ref[0, 0] = prev + loss * (1.0 / B)


def _eval(skeys):
    out = pl.pallas_call(
        _eval_body,
        grid=(B,),
        in_specs=[pl.BlockSpec((1, ROWS, LANES), lambda b: (b, 0, 0))],
        out_specs=pl.BlockSpec((1, 1), lambda b: (0, 0),
                               memory_space=pltpu.SMEM),
        out_shape=jax.ShapeDtypeStruct((1, 1), jnp.float32),
    )(skeys)
    return out.reshape(())


@jax.jit
def _run(logits, target):
    lg = logits.reshape(B, ROWS, LANES)
    tg = target.reshape(B, ROWS, LANES)
    keys = _pack(lg, tg).reshape(B, P)
    skeys = _sc_sort(keys)
    return _eval(skeys.reshape(B, ROWS, LANES))


def kernel(logits, target):
    return _run(logits, target)


# speedup vs baseline: 1.3375x; 1.0143x over previous
import functools

import jax
import jax.numpy as jnp
import numpy as np
from jax import lax
from jax.experimental import pallas as pl
from jax.experimental.pallas import tpu as pltpu
from jax.experimental.pallas import tpu_sc as plsc

ROWS, LANES = 2048, 128
P = ROWS * LANES
B = 8
NT = 16
CHUNK = P // NT
NV = CHUNK // 16
IMGS_PER_CORE = 4


def _pack_body(logits_ref, target_ref, keys_ref):
    lab = target_ref[0]
    labf = lab.astype(jnp.float32)
    e = 1.0 - logits_ref[0] * (2.0 * labf - 1.0)
    epos = jnp.maximum(e, 0.0)
    bits = (lax.bitcast_convert_type(epos, jnp.int32) & jnp.int32(~1)) | lab
    keys_ref[0] = bits


def _pack(lg, tg):
    return pl.pallas_call(
        _pack_body,
        grid=(B,),
        in_specs=[
            pl.BlockSpec((1, ROWS, LANES), lambda b: (b, 0, 0)),
            pl.BlockSpec((1, ROWS, LANES), lambda b: (b, 0, 0)),
        ],
        out_specs=pl.BlockSpec((1, ROWS, LANES), lambda b: (b, 0, 0)),
        out_shape=jax.ShapeDtypeStruct((B, ROWS, LANES), jnp.int32),
    )(lg, tg)


def _take16(vec, idx):
    return lax.gather(
        vec, idx[:, None],
        dimension_numbers=lax.GatherDimensionNumbers(
            offset_dims=(), collapsed_slice_dims=(0,), start_index_map=(0,)),
        slice_sizes=(1,),
        mode=lax.GatherScatterMode.PROMISE_IN_BOUNDS)


def _sc_sort(keys):
    mesh = plsc.VectorSubcoreMesh(core_axis_name="c", subcore_axis_name="s")

    NB = 2048
    PH = 256

    @functools.partial(
        pl.kernel,
        mesh=mesh,
        compiler_params=pltpu.CompilerParams(use_tc_tiling_on_sc=False,
                                             needs_layout_passes=False),
        out_type=jax.ShapeDtypeStruct((B, P), jnp.int32),
        scratch_types=[
            pltpu.VMEM((CHUNK,), jnp.int32),
            pltpu.VMEM((CHUNK,), jnp.int32),
            pltpu.VMEM((16, NB), jnp.int32),
            pltpu.VMEM((NB,), jnp.int32),
            pltpu.VMEM((NT, PH), jnp.int32),
            pltpu.VMEM_SHARED((P,), jnp.int32),
            pltpu.VMEM_SHARED((P,), jnp.int32),
            pltpu.VMEM_SHARED((NT, PH), jnp.int32),
            pltpu.SemaphoreType.DMA,
        ],
    )
    def k(keys_hbm, out_hbm, buf, posbuf, tbl,
          hist, hists_local, S0, S1, hist_sh, sem):
        c = lax.axis_index("c")
        t = lax.axis_index("s")
        lane = lax.iota(jnp.int32, 16)
        zeros16 = jnp.zeros((16,), jnp.int32)
        my = pl.ds(t * CHUNK, CHUNK)

        def zero_tbl(ch, _):
            chs = pl.ds(ch * 16, 16)
            for l in range(16):
                tbl[l, chs] = zeros16
            return 0

        lax.fori_loop(0, NB // 16, zero_tbl, 0)

        def do_pass(shift, nbits, dst, last=False):
            nb = 1 << nbits
            comp = nb - 1

            def sweep(q, _):
                sl = pl.ds(q * 16, 16)
                d = comp - ((buf[sl] >> shift) & comp)
                cg = plsc.load_gather(tbl, [lane, d])
                plsc.store_scatter(tbl, [lane, d], cg + 1)
                posbuf[sl] = cg
                return 0

            lax.fori_loop(0, NV, sweep, 0)

            def pfx_chunk(ch, _):
                chs = pl.ds(ch * 16, 16)
                acc = zeros16
                for l in range(16):
                    rowv = tbl[l, chs]
                    tbl[l, chs] = acc
                    acc = acc + rowv
                hist[chs] = acc
                return 0

            lax.fori_loop(0, nb // 16, pfx_chunk, 0)

            def phase(ph, carry):
                pltpu.sync_copy(hist.at[pl.ds(ph * PH, PH)], hist_sh.at[t])
                plsc.subcore_barrier()
                pltpu.sync_copy(hist_sh, hists_local)

                def scan_chunk(ch, carry):
                    col = zeros16
                    part = zeros16
                    for tt in range(16):
                        h = hists_local[tt, pl.ds(ch * 16, 16)]
                        col = col + h
                        tv = jnp.full((16,), tt, jnp.int32)
                        part = part + jnp.where(tv < t, h, zeros16)
                    incl = plsc.cumsum(col)
                    ov = (incl - col) + carry + part
                    chs = pl.ds(ph * PH + ch * 16, 16)
                    for l in range(16):
                        tbl[l, chs] = tbl[l, chs] + ov
                    return carry + jnp.sum(col)

                carry = lax.fori_loop(0, PH // 16, scan_chunk, carry)
                plsc.subcore_barrier()
                return carry

            lax.fori_loop(0, nb // PH, phase, jnp.int32(0))

            def fix(q, _):
                sl = pl.ds(q * 16, 16)
                dv = comp - ((buf[sl] >> shift) & comp)
                pos = posbuf[sl] + plsc.load_gather(tbl, [lane, dv])
                if not last:
                    pos = ((pos & ~jnp.int32(16383)) | ((pos & 1023) << 4)
                           | ((pos >> 10) & 15))
                posbuf[sl] = pos
                return 0

            lax.fori_loop(0, NV, fix, 0)
            cp = pltpu.async_copy(buf, dst.at[posbuf], sem)

            def rezero(ch, _):
                chs = pl.ds(ch * 16, 16)
                for l in range(16):
                    tbl[l, chs] = zeros16
                return 0

            lax.fori_loop(0, nb // 16, rezero, 0)
            cp.wait()
            plsc.subcore_barrier()

        def img_body(ii, _):
            img = c * IMGS_PER_CORE + ii
            pltpu.sync_copy(keys_hbm.at[img, my], buf)
            do_pass(0, 11, S0)
            pltpu.sync_copy(S0.at[my], buf)
            do_pass(11, 10, S1)
            pltpu.sync_copy(S1.at[my], buf)
            do_pass(21, 10, S0, last=True)
            pltpu.sync_copy(S0.at[my], out_hbm.at[img, my])
            plsc.subcore_barrier()
            return 0

        lax.fori_loop(0, IMGS_PER_CORE, img_body, 0)

    return k(keys)


def _eval_body(skeys_ref, out_ref):
    b = pl.program_id(0)
    row = lax.broadcasted_iota(jnp.int32, (ROWS, LANES), 0)
    lane = lax.broadcasted_iota(jnp.int32, (ROWS, LANES), 1)

    sbits = skeys_ref[0]
    l_sorted = (sbits & 1).astype(jnp.float32)
    e_sorted = lax.bitcast_convert_type(sbits & jnp.int32(~1), jnp.float32)
    G = jnp.sum(l_sorted)

    cs = l_sorted
    for sh in (1, 2, 4, 8, 16, 32, 64):
        cs = cs + jnp.where(lane >= sh, pltpu.roll(cs, sh, axis=1), 0.0)
    rt = cs[:, LANES - 1:LANES]
    rs = rt
    rowv = lax.broadcasted_iota(jnp.int32, (ROWS, 1), 0)
    for sh in (1, 2, 4, 8, 16, 32, 64, 128, 256, 512, 1024):
        rs = rs + jnp.where(rowv >= sh, pltpu.roll(rs, sh, axis=0), 0.0)
    c = cs + (rs - rt)

    k = (row * LANES + lane).astype(jnp.float32) + 1.0
    n = k - c
    cm1 = c - l_sorted
    nm1 = (k - 1.0) - cm1
    jk = k / (G + n)
    jm1 = (k - 1.0) / jnp.maximum(G + nm1, 1.0)
    loss = jnp.sum(e_sorted * (jk - jm1))

    prev = jnp.where(b == 0, 0.0, out_ref[0, 0])
    out_ref[0, 0] = prev + loss * (1.0 / B)


def _eval(skeys):
    out = pl.pallas_call(
        _eval_body,
        grid=(B,),
        in_specs=[pl.BlockSpec((1, ROWS, LANES), lambda b: (b, 0, 0))],
        out_specs=pl.BlockSpec((1, 1), lambda b: (0, 0),
                               memory_space=pltpu.SMEM),
        out_shape=jax.ShapeDtypeStruct((1, 1), jnp.float32),
    )(skeys)
    return out.reshape(())


@jax.jit
def _run(logits, target):
    lg = logits.reshape(B, ROWS, LANES)
    tg = target.reshape(B, ROWS, LANES)
    keys = _pack(lg, tg).reshape(B, P)
    skeys = _sc_sort(keys)
    return _eval(skeys.reshape(B, ROWS, LANES))


def kernel(logits, target):
    return _run(logits, target)


# speedup vs baseline: 1.4449x; 1.0803x over previous
import functools

import jax
import jax.numpy as jnp
import numpy as np
from jax import lax
from jax.experimental import pallas as pl
from jax.experimental.pallas import tpu as pltpu
from jax.experimental.pallas import tpu_sc as plsc

ROWS, LANES = 2048, 128
P = ROWS * LANES
B = 8
NT = 16
CHUNK = P // NT
NV = CHUNK // 16
IMGS_PER_CORE = 4


def _pack_body(logits_ref, target_ref, keys_ref):
    lab = target_ref[0]
    labf = lab.astype(jnp.float32)
    e = 1.0 - logits_ref[0] * (2.0 * labf - 1.0)
    epos = jnp.maximum(e, 0.0)
    bits = (lax.bitcast_convert_type(epos, jnp.int32) & jnp.int32(~1)) | lab
    keys_ref[0] = bits


def _pack(lg, tg):
    return pl.pallas_call(
        _pack_body,
        grid=(B,),
        in_specs=[
            pl.BlockSpec((1, ROWS, LANES), lambda b: (b, 0, 0)),
            pl.BlockSpec((1, ROWS, LANES), lambda b: (b, 0, 0)),
        ],
        out_specs=pl.BlockSpec((1, ROWS, LANES), lambda b: (b, 0, 0)),
        out_shape=jax.ShapeDtypeStruct((B, ROWS, LANES), jnp.int32),
    )(lg, tg)


def _take16(vec, idx):
    return lax.gather(
        vec, idx[:, None],
        dimension_numbers=lax.GatherDimensionNumbers(
            offset_dims=(), collapsed_slice_dims=(0,), start_index_map=(0,)),
        slice_sizes=(1,),
        mode=lax.GatherScatterMode.PROMISE_IN_BOUNDS)


def _sc_sort(keys):
    mesh = plsc.VectorSubcoreMesh(core_axis_name="c", subcore_axis_name="s")

    NB = 2048
    PH = 256

    @functools.partial(
        pl.kernel,
        mesh=mesh,
        compiler_params=pltpu.CompilerParams(use_tc_tiling_on_sc=False,
                                             needs_layout_passes=False),
        out_type=jax.ShapeDtypeStruct((B, P), jnp.int32),
        scratch_types=[
            pltpu.VMEM((CHUNK,), jnp.int32),
            pltpu.VMEM((CHUNK,), jnp.int32),
            pltpu.VMEM((16, NB), jnp.int32),
            pltpu.VMEM((NB,), jnp.int32),
            pltpu.VMEM((NT, PH), jnp.int32),
            pltpu.VMEM_SHARED((P,), jnp.int32),
            pltpu.VMEM_SHARED((P,), jnp.int32),
            pltpu.VMEM_SHARED((NT, PH), jnp.int32),
            pltpu.SemaphoreType.DMA,
        ],
    )
    def k(keys_hbm, out_hbm, buf, posbuf, tbl,
          hist, hists_local, S0, S1, hist_sh, sem):
        c = lax.axis_index("c")
        t = lax.axis_index("s")
        lane = lax.iota(jnp.int32, 16)
        zeros16 = jnp.zeros((16,), jnp.int32)
        my = pl.ds(t * CHUNK, CHUNK)

        def zero_tbl(ch, _):
            chs = pl.ds(ch * 16, 16)
            for l in range(16):
                tbl[l, chs] = zeros16
            return 0

        lax.fori_loop(0, NB // 16, zero_tbl, 0)

        def do_pass(shift, nbits, dst, last=False):
            nb = 1 << nbits
            comp = nb - 1

            def sweep(q, _):
                sl = pl.ds(q * 16, 16)
                d = comp - ((buf[sl] >> shift) & comp)
                cg = plsc.load_gather(tbl, [lane, d])
                plsc.store_scatter(tbl, [lane, d], cg + 1)
                posbuf[sl] = cg
                return 0

            lax.fori_loop(0, NV, sweep, 0)

            def pfx_chunk(ch, _):
                chs = pl.ds(ch * 16, 16)
                acc = zeros16
                for l in range(16):
                    rowv = tbl[l, chs]
                    tbl[l, chs] = acc
                    acc = acc + rowv
                hist[chs] = acc
                return 0

            lax.fori_loop(0, nb // 16, pfx_chunk, 0)

            def phase(ph, carry):
                pltpu.sync_copy(hist.at[pl.ds(ph * PH, PH)], hist_sh.at[t])
                plsc.subcore_barrier()
                pltpu.sync_copy(hist_sh, hists_local)

                def scan_chunk(ch, carry):
                    col = zeros16
                    part = zeros16
                    for tt in range(16):
                        h = hists_local[tt, pl.ds(ch * 16, 16)]
                        col = col + h
                        tv = jnp.full((16,), tt, jnp.int32)
                        part = part + jnp.where(tv < t, h, zeros16)
                    incl = plsc.cumsum(col)
                    ov = (incl - col) + carry + part
                    chs = pl.ds(ph * PH + ch * 16, 16)
                    for l in range(16):
                        tbl[l, chs] = tbl[l, chs] + ov
                    return carry + jnp.sum(col)

                carry = lax.fori_loop(0, PH // 16, scan_chunk, carry)
                plsc.subcore_barrier()
                return carry

            lax.fori_loop(0, nb // PH, phase, jnp.int32(0))

            def fix(q, _):
                sl = pl.ds(q * 16, 16)
                dv = comp - ((buf[sl] >> shift) & comp)
                pos = posbuf[sl] + plsc.load_gather(tbl, [lane, dv])
                if not last:
                    pos = ((pos & ~jnp.int32(16383)) | ((pos & 1023) << 4)
                           | ((pos >> 10) & 15))
                posbuf[sl] = pos
                return 0

            lax.fori_loop(0, NV, fix, 0)
            cp = pltpu.async_copy(buf, dst.at[posbuf], sem)

            def rezero(ch, _):
                chs = pl.ds(ch * 16, 16)
                for l in range(16):
                    tbl[l, chs] = zeros16
                return 0

            lax.fori_loop(0, nb // 16, rezero, 0)
            cp.wait()
            plsc.subcore_barrier()

        def img_body(ii, _):
            img = c * IMGS_PER_CORE + ii
            pltpu.sync_copy(keys_hbm.at[img, my], buf)
            do_pass(0, 11, S0)
            pltpu.sync_copy(S0.at[my], buf)
            do_pass(11, 10, S1)
            pltpu.sync_copy(S1.at[my], buf)
            do_pass(21, 10, S0, last=True)
            pltpu.sync_copy(S0.at[my], out_hbm.at[img, my])
            plsc.subcore_barrier()
            return 0

        lax.fori_loop(0, IMGS_PER_CORE, img_body, 0)

    return k(keys)


def _eval_body(skeys_ref, out_ref):
    b = pl.program_id(0)
    row = lax.broadcasted_iota(jnp.int32, (ROWS, LANES), 0)
    lane = lax.broadcasted_iota(jnp.int32, (ROWS, LANES), 1)

    sbits = skeys_ref[0]
    l_sorted = (sbits & 1).astype(jnp.float32)
    e_sorted = lax.bitcast_convert_type(sbits & jnp.int32(~1), jnp.float32)
    G = jnp.sum(l_sorted)

    ur = lax.broadcasted_iota(jnp.int32, (LANES, LANES), 0)
    uc = lax.broadcasted_iota(jnp.int32, (LANES, LANES), 1)
    tri = (ur <= uc).astype(jnp.float32)
    cs = jax.lax.dot_general(l_sorted, tri, (((1,), (0,)), ((), ())),
                             preferred_element_type=jnp.float32)
    l3 = l_sorted.reshape(16, LANES, LANES)
    tot16 = jnp.sum(l3, axis=2)
    inc16 = jax.lax.dot_general(tot16, tri, (((1,), (0,)), ((), ())),
                                preferred_element_type=jnp.float32)
    gt = inc16[:, LANES - 1:LANES]
    g_inc = gt
    rowv = lax.broadcasted_iota(jnp.int32, (16, 1), 0)
    for sh in (1, 2, 4, 8):
        g_inc = g_inc + jnp.where(rowv >= sh, pltpu.roll(g_inc, sh, axis=0),
                                  0.0)
    pre16 = inc16 - tot16 + (g_inc - gt)
    rp = jnp.broadcast_to(pre16[:, :, None],
                          (16, LANES, LANES)).reshape(ROWS, LANES)
    c = cs + rp

    k = (row * LANES + lane).astype(jnp.float32) + 1.0
    n = k - c
    cm1 = c - l_sorted
    nm1 = (k - 1.0) - cm1
    jk = k / (G + n)
    jm1 = (k - 1.0) / jnp.maximum(G + nm1, 1.0)
    loss = jnp.sum(e_sorted * (jk - jm1))

    prev = jnp.where(b == 0, 0.0, out_ref[0, 0])
    out_ref[0, 0] = prev + loss * (1.0 / B)


def _eval(skeys):
    out = pl.pallas_call(
        _eval_body,
        grid=(B,),
        in_specs=[pl.BlockSpec((1, ROWS, LANES), lambda b: (b, 0, 0))],
        out_specs=pl.BlockSpec((1, 1), lambda b: (0, 0),
                               memory_space=pltpu.SMEM),
        out_shape=jax.ShapeDtypeStruct((1, 1), jnp.float32),
    )(skeys)
    return out.reshape(())


@jax.jit
def _run(logits, target):
    lg = logits.reshape(B, ROWS, LANES)
    tg = target.reshape(B, ROWS, LANES)
    keys = _pack(lg, tg).reshape(B, P)
    skeys = _sc_sort(keys)
    return _eval(skeys.reshape(B, ROWS, LANES))


def kernel(logits, target):
    return _run(logits, target)
